# baseline (speedup 1.0000x reference)
import functools

import jax
import jax.numpy as jnp
from jax import lax
from jax.experimental import pallas as pl
from jax.experimental.pallas import tpu as pltpu
from jax.experimental.pallas import tpu_sc as plsc

N = 10000
E = 320000
DF = 128
NSUB = 16
L = 16
SLICE = 640
NPAD = NSUB * SLICE
BUFE = 10240
COLS = 39


def _worker_geometry(cid, sid):
    col0 = sid * COLS + cid * 20
    n16 = jnp.where(cid == 0, 640, jnp.where(sid == NSUB - 1, 640, 608))
    return col0 * 512, n16


def _zero_slice(sh_ref, zbuf, sid):
    zeros = jnp.zeros((L,), jnp.float32)

    @plsc.parallel_loop(0, SLICE, L, unroll=8)
    def _z(i):
        zbuf[pl.ds(i, L)] = zeros
    pltpu.sync_copy(zbuf, sh_ref.at[pl.ds(sid * SLICE, SLICE)])


def _sc1_body(ei_hbm, degp_out, ebuf2, ebuf_s, vbuf, zbuf, deg_sh, sem):
    cid = lax.axis_index("c")
    sid = lax.axis_index("s")
    eoff, cnt16 = _worker_geometry(cid, sid)
    dma = pltpu.async_copy(ei_hbm.at[:, pl.ds(eoff, BUFE)], ebuf2, sem)
    _zero_slice(deg_sh, zbuf, sid)

    live = cnt16 * L

    @plsc.parallel_loop(0, BUFE, L, unroll=8)
    def _o(i):
        vbuf[pl.ds(i, L)] = jnp.where(i < live, 1.0, 0.0).astype(
            jnp.float32) + jnp.zeros((L,), jnp.float32)

    dma.wait()

    @plsc.parallel_loop(0, BUFE, L, unroll=8)
    def _f(i):
        ebuf_s[pl.ds(i, L)] = ebuf2[0, pl.ds(i, L)]
    plsc.subcore_barrier()
    pltpu.sync_copy(vbuf, deg_sh.at[ebuf_s], add=True)
    plsc.subcore_barrier()
    off = sid * SLICE
    pltpu.sync_copy(deg_sh.at[pl.ds(off, SLICE)], zbuf)
    pltpu.sync_copy(zbuf, degp_out.at[pl.ds(cid * NPAD + off, SLICE)])


def _sc2_body(ei_hbm, degp_hbm, c0_out, c1_out, t_out,
              ebuf2, ebuf_s, vbuf, abuf, bbuf, cvbuf, cibuf, tbuf,
              c_sh, cand_sh, sem):
    cid = lax.axis_index("c")
    sid = lax.axis_index("s")
    iota_f = lax.iota(jnp.int32, L).astype(jnp.float32)

    eoff, cnt16 = _worker_geometry(cid, sid)
    dma = pltpu.async_copy(ei_hbm.at[:, pl.ds(eoff, BUFE)], ebuf2, sem)
    _zero_slice(c_sh, abuf, sid)

    off = sid * SLICE
    pltpu.sync_copy(degp_hbm.at[pl.ds(off, SLICE)], abuf)
    pltpu.sync_copy(degp_hbm.at[pl.ds(NPAD + off, SLICE)], bbuf)
    base_f = off.astype(jnp.float32)
    bv0 = abuf[pl.ds(0, L)] + bbuf[pl.ds(0, L)]
    bi0 = base_f + iota_f

    def _scan(i, carry):
        bv, bi = carry
        v = abuf[pl.ds(i * L, L)] + bbuf[pl.ds(i * L, L)]
        idx = base_f + (i * L).astype(jnp.float32) + iota_f
        upd = v > bv
        return (jnp.where(upd, v, bv), jnp.where(upd, idx, bi))

    bv, bi = lax.fori_loop(1, SLICE // L, _scan, (bv0, bi0))
    cvbuf[pl.ds(0, L)] = bv
    cibuf[pl.ds(0, L)] = bi
    pltpu.sync_copy(cvbuf.at[pl.ds(0, L)], cand_sh.at[pl.ds(sid * L, L)])
    pltpu.sync_copy(cibuf.at[pl.ds(0, L)],
                    cand_sh.at[pl.ds(NSUB * L + sid * L, L)])
    plsc.subcore_barrier()

    pltpu.sync_copy(cand_sh.at[pl.ds(0, NSUB * L)], cvbuf)
    pltpu.sync_copy(cand_sh.at[pl.ds(NSUB * L, NSUB * L)], cibuf)
    rv0 = cvbuf[pl.ds(0, L)]
    ri0 = cibuf[pl.ds(0, L)]

    def _red(w, carry):
        bv_, bi_ = carry
        v = cvbuf[pl.ds(w * L, L)]
        ii = cibuf[pl.ds(w * L, L)]
        take = (v > bv_) | ((v == bv_) & (ii < bi_))
        return (jnp.where(take, v, bv_), jnp.where(take, ii, bi_))

    rv, ri = lax.fori_loop(1, NSUB, _red, (rv0, ri0))
    iota_i = lax.iota(jnp.int32, L)
    for k in (1, 2, 4, 8):
        cvbuf[pl.ds(0, L)] = rv
        cibuf[pl.ds(0, L)] = ri
        perm = iota_i ^ k
        ov = plsc.load_gather(cvbuf.at[pl.ds(0, L)], [perm])
        oi = plsc.load_gather(cibuf.at[pl.ds(0, L)], [perm])
        take = (ov > rv) | ((ov == rv) & (oi < ri))
        rv = jnp.where(take, ov, rv)
        ri = jnp.where(take, oi, ri)
    tvec = ri.astype(jnp.int32)

    @pl.when((sid == 0) & (cid == 0))
    def _():
        tbuf[...] = tvec
        pltpu.sync_copy(tbuf, t_out)

    dma.wait()
    live = cnt16 * L

    for j in range(BUFE // 512):
        zer = jnp.zeros((L,), jnp.int32)

        @plsc.parallel_loop(j * 512, (j + 1) * 512, L, unroll=8, carry=zer)
        def _cmp(i, acc):
            ebuf_s[pl.ds(i, L)] = ebuf2[0, pl.ds(i, L)]
            d = ebuf2[1, pl.ds(i, L)]
            m = (d == tvec) & (i < live)
            vbuf[pl.ds(i, L)] = jnp.where(m, 1.0, 0.0).astype(jnp.float32)
            return acc | m.astype(jnp.int32)

        nmatch = plsc.all_reduce_population_count(_cmp != 0)

        @pl.when(nmatch[0] > 0)
        def _():
            pltpu.sync_copy(vbuf.at[pl.ds(j * 512, 512)],
                            c_sh.at[ebuf_s.at[pl.ds(j * 512, 512)]],
                            add=True)

    plsc.subcore_barrier()

    off = sid * SLICE
    pltpu.sync_copy(c_sh.at[pl.ds(off, SLICE)], abuf)

    @pl.when(cid == 0)
    def _():
        pltpu.sync_copy(abuf, c0_out.at[pl.ds(off, SLICE)])

    @pl.when(cid == 1)
    def _():
        pltpu.sync_copy(abuf, c1_out.at[pl.ds(off, SLICE)])


def _sc_counts(ei):
    mesh = plsc.VectorSubcoreMesh(core_axis_name="c", subcore_axis_name="s")
    k1 = pl.kernel(
        _sc1_body,
        out_type=[jax.ShapeDtypeStruct((2 * NPAD,), jnp.float32)],
        mesh=mesh,
        compiler_params=pltpu.CompilerParams(needs_layout_passes=False),
        scratch_types=[
            pltpu.VMEM((2, BUFE), jnp.int32),
            pltpu.VMEM((BUFE,), jnp.int32),
            pltpu.VMEM((BUFE,), jnp.float32),
            pltpu.VMEM((SLICE,), jnp.float32),
            pltpu.VMEM_SHARED((NPAD,), jnp.float32),
            pltpu.SemaphoreType.DMA,
        ],
    )
    (degp,) = k1(ei)
    k2 = pl.kernel(
        _sc2_body,
        out_type=[
            jax.ShapeDtypeStruct((NPAD,), jnp.float32),
            jax.ShapeDtypeStruct((NPAD,), jnp.float32),
            jax.ShapeDtypeStruct((L,), jnp.int32),
        ],
        mesh=mesh,
        compiler_params=pltpu.CompilerParams(needs_layout_passes=False),
        scratch_types=[
            pltpu.VMEM((2, BUFE), jnp.int32),
            pltpu.VMEM((BUFE,), jnp.int32),
            pltpu.VMEM((BUFE,), jnp.float32),
            pltpu.VMEM((SLICE,), jnp.float32),
            pltpu.VMEM((SLICE,), jnp.float32),
            pltpu.VMEM((NSUB * L,), jnp.float32),
            pltpu.VMEM((NSUB * L,), jnp.float32),
            pltpu.VMEM((L,), jnp.int32),
            pltpu.VMEM_SHARED((NPAD,), jnp.float32),
            pltpu.VMEM_SHARED((2 * NSUB * L,), jnp.float32),
            pltpu.SemaphoreType.DMA,
        ],
    )
    return k2(ei, degp)


def _tc_body(t_ref, c0_ref, c1_ref, x_ref, w_ref, b_ref, o_ref):
    h8 = lax.dot_general(x_ref[...], w_ref[...], (((1,), (1,)), ((), ())),
                         preferred_element_type=jnp.float32)
    r = t_ref[0] % 8
    rmask = lax.broadcasted_iota(jnp.int32, (8, 1), 0) == r
    h = jnp.sum(jnp.where(rmask, h8, 0.0), axis=0, keepdims=True) + b_ref[...]
    c = c0_ref[...] + c1_ref[...]
    o_ref[...] = lax.dot_general(c, h, (((0,), (0,)), ((), ())),
                                 preferred_element_type=jnp.float32)


def _tc_outer(c0, c1, x, w, b2, t_sp):
    blk = 5120
    cspec = pl.BlockSpec((1, blk), lambda i, t_ref: (0, i))
    grid_spec = pltpu.PrefetchScalarGridSpec(
        num_scalar_prefetch=1,
        grid=(NPAD // blk,),
        in_specs=[
            cspec,
            cspec,
            pl.BlockSpec((8, DF), lambda i, t_ref: (t_ref[0] // 8, 0)),
            pl.BlockSpec((DF, DF), lambda i, t_ref: (0, 0)),
            pl.BlockSpec((1, DF), lambda i, t_ref: (0, 0)),
        ],
        out_specs=pl.BlockSpec((blk, DF), lambda i, t_ref: (i, 0)),
    )
    return pl.pallas_call(
        _tc_body,
        grid_spec=grid_spec,
        out_shape=jax.ShapeDtypeStruct((N, DF), jnp.float32),
    )(t_sp, c0, c1, x, w, b2)


def kernel(x, edge_index, batch_index, W, b):
    c0, c1, t_vec = _sc_counts(edge_index)
    return _tc_outer(c0.reshape(1, NPAD), c1.reshape(1, NPAD), x, W,
                     b.reshape(1, DF), t_vec[:1])


# speedup vs baseline: 1.0194x; 1.0194x over previous
import functools

import jax
import jax.numpy as jnp
from jax import lax
from jax.experimental import pallas as pl
from jax.experimental.pallas import tpu as pltpu
from jax.experimental.pallas import tpu_sc as plsc

N = 10000
E = 320000
DF = 128
NSUB = 16
L = 16
SLICE = 640
NPAD = NSUB * SLICE
BUFE = 10240
COLS = 39


def _worker_geometry(cid, sid):
    col0 = sid * COLS + cid * 20
    n16 = jnp.where(cid == 0, 640, jnp.where(sid == NSUB - 1, 640, 608))
    return col0 * 512, n16


def _zero_slice(sh_ref, zbuf, sid):
    zeros = jnp.zeros((L,), jnp.float32)

    @plsc.parallel_loop(0, SLICE, L, unroll=8)
    def _z(i):
        zbuf[pl.ds(i, L)] = zeros
    pltpu.sync_copy(zbuf, sh_ref.at[pl.ds(sid * SLICE, SLICE)])


def _sc1_body(ei_hbm, degp_out, ebuf2, ebuf_s, vbuf, zbuf, deg_sh, sem):
    cid = lax.axis_index("c")
    sid = lax.axis_index("s")
    eoff, cnt16 = _worker_geometry(cid, sid)
    dma = pltpu.async_copy(ei_hbm.at[:, pl.ds(eoff, BUFE)], ebuf2, sem)
    _zero_slice(deg_sh, zbuf, sid)

    live = cnt16 * L

    @plsc.parallel_loop(0, BUFE, L, unroll=8)
    def _o(i):
        vbuf[pl.ds(i, L)] = jnp.where(i < live, 1.0, 0.0).astype(
            jnp.float32) + jnp.zeros((L,), jnp.float32)

    dma.wait()

    @plsc.parallel_loop(0, BUFE, L, unroll=8)
    def _f(i):
        ebuf_s[pl.ds(i, L)] = ebuf2[0, pl.ds(i, L)]
    plsc.subcore_barrier()
    pltpu.sync_copy(vbuf, deg_sh.at[ebuf_s], add=True)
    plsc.subcore_barrier()
    off = sid * SLICE
    pltpu.sync_copy(deg_sh.at[pl.ds(off, SLICE)], zbuf)
    pltpu.sync_copy(zbuf, degp_out.at[pl.ds(cid * NPAD + off, SLICE)])


def _sc2_body(ei_hbm, degp_hbm, c0_out, c1_out, t_out,
              ebuf2, ebuf_s, vbuf, abuf, bbuf, cvbuf, cibuf, tbuf,
              c_sh, cand_sh, sem):
    cid = lax.axis_index("c")
    sid = lax.axis_index("s")
    iota_f = lax.iota(jnp.int32, L).astype(jnp.float32)

    eoff, cnt16 = _worker_geometry(cid, sid)
    dma = pltpu.async_copy(ei_hbm.at[:, pl.ds(eoff, BUFE)], ebuf2, sem)
    _zero_slice(c_sh, abuf, sid)

    off = sid * SLICE
    pltpu.sync_copy(degp_hbm.at[pl.ds(off, SLICE)], abuf)
    pltpu.sync_copy(degp_hbm.at[pl.ds(NPAD + off, SLICE)], bbuf)
    base_f = off.astype(jnp.float32)
    bv0 = abuf[pl.ds(0, L)] + bbuf[pl.ds(0, L)]
    bi0 = base_f + iota_f

    def _scan(i, carry):
        bv, bi = carry
        v = abuf[pl.ds(i * L, L)] + bbuf[pl.ds(i * L, L)]
        idx = base_f + (i * L).astype(jnp.float32) + iota_f
        upd = v > bv
        return (jnp.where(upd, v, bv), jnp.where(upd, idx, bi))

    bv, bi = lax.fori_loop(1, SLICE // L, _scan, (bv0, bi0))
    cvbuf[pl.ds(0, L)] = bv
    cibuf[pl.ds(0, L)] = bi
    pltpu.sync_copy(cvbuf.at[pl.ds(0, L)], cand_sh.at[pl.ds(sid * L, L)])
    pltpu.sync_copy(cibuf.at[pl.ds(0, L)],
                    cand_sh.at[pl.ds(NSUB * L + sid * L, L)])
    plsc.subcore_barrier()

    pltpu.sync_copy(cand_sh.at[pl.ds(0, NSUB * L)], cvbuf)
    pltpu.sync_copy(cand_sh.at[pl.ds(NSUB * L, NSUB * L)], cibuf)
    rv0 = cvbuf[pl.ds(0, L)]
    ri0 = cibuf[pl.ds(0, L)]

    def _red(w, carry):
        bv_, bi_ = carry
        v = cvbuf[pl.ds(w * L, L)]
        ii = cibuf[pl.ds(w * L, L)]
        take = (v > bv_) | ((v == bv_) & (ii < bi_))
        return (jnp.where(take, v, bv_), jnp.where(take, ii, bi_))

    rv, ri = lax.fori_loop(1, NSUB, _red, (rv0, ri0))
    iota_i = lax.iota(jnp.int32, L)
    for k in (1, 2, 4, 8):
        cvbuf[pl.ds(0, L)] = rv
        cibuf[pl.ds(0, L)] = ri
        perm = iota_i ^ k
        ov = plsc.load_gather(cvbuf.at[pl.ds(0, L)], [perm])
        oi = plsc.load_gather(cibuf.at[pl.ds(0, L)], [perm])
        take = (ov > rv) | ((ov == rv) & (oi < ri))
        rv = jnp.where(take, ov, rv)
        ri = jnp.where(take, oi, ri)
    tvec = ri.astype(jnp.int32)

    @pl.when((sid == 0) & (cid == 0))
    def _():
        tbuf[...] = tvec
        pltpu.sync_copy(tbuf, t_out)

    dma.wait()
    live = cnt16 * L

    @plsc.parallel_loop(0, BUFE, L, unroll=8)
    def _cmp(i):
        ebuf_s[pl.ds(i, L)] = ebuf2[0, pl.ds(i, L)]
        d = ebuf2[1, pl.ds(i, L)]
        vbuf[pl.ds(i, L)] = jnp.where(
            (d == tvec) & (i < live), 1.0, 0.0).astype(jnp.float32)

    pltpu.sync_copy(vbuf, c_sh.at[ebuf_s], add=True)
    plsc.subcore_barrier()

    off = sid * SLICE
    pltpu.sync_copy(c_sh.at[pl.ds(off, SLICE)], abuf)

    @pl.when(cid == 0)
    def _():
        pltpu.sync_copy(abuf, c0_out.at[pl.ds(off, SLICE)])

    @pl.when(cid == 1)
    def _():
        pltpu.sync_copy(abuf, c1_out.at[pl.ds(off, SLICE)])


def _sc_counts(ei):
    mesh = plsc.VectorSubcoreMesh(core_axis_name="c", subcore_axis_name="s")
    k1 = pl.kernel(
        _sc1_body,
        out_type=[jax.ShapeDtypeStruct((2 * NPAD,), jnp.float32)],
        mesh=mesh,
        compiler_params=pltpu.CompilerParams(needs_layout_passes=False),
        scratch_types=[
            pltpu.VMEM((2, BUFE), jnp.int32),
            pltpu.VMEM((BUFE,), jnp.int32),
            pltpu.VMEM((BUFE,), jnp.float32),
            pltpu.VMEM((SLICE,), jnp.float32),
            pltpu.VMEM_SHARED((NPAD,), jnp.float32),
            pltpu.SemaphoreType.DMA,
        ],
    )
    (degp,) = k1(ei)
    k2 = pl.kernel(
        _sc2_body,
        out_type=[
            jax.ShapeDtypeStruct((NPAD,), jnp.float32),
            jax.ShapeDtypeStruct((NPAD,), jnp.float32),
            jax.ShapeDtypeStruct((L,), jnp.int32),
        ],
        mesh=mesh,
        compiler_params=pltpu.CompilerParams(needs_layout_passes=False),
        scratch_types=[
            pltpu.VMEM((2, BUFE), jnp.int32),
            pltpu.VMEM((BUFE,), jnp.int32),
            pltpu.VMEM((BUFE,), jnp.float32),
            pltpu.VMEM((SLICE,), jnp.float32),
            pltpu.VMEM((SLICE,), jnp.float32),
            pltpu.VMEM((NSUB * L,), jnp.float32),
            pltpu.VMEM((NSUB * L,), jnp.float32),
            pltpu.VMEM((L,), jnp.int32),
            pltpu.VMEM_SHARED((NPAD,), jnp.float32),
            pltpu.VMEM_SHARED((2 * NSUB * L,), jnp.float32),
            pltpu.SemaphoreType.DMA,
        ],
    )
    return k2(ei, degp)


def _tc_body(t_ref, c0_ref, c1_ref, x_ref, w_ref, b_ref, o_ref):
    h8 = lax.dot_general(x_ref[...], w_ref[...], (((1,), (1,)), ((), ())),
                         preferred_element_type=jnp.float32)
    r = t_ref[0] % 8
    rmask = lax.broadcasted_iota(jnp.int32, (8, 1), 0) == r
    h = jnp.sum(jnp.where(rmask, h8, 0.0), axis=0, keepdims=True) + b_ref[...]
    c = c0_ref[...] + c1_ref[...]
    o_ref[...] = lax.dot_general(c, h, (((0,), (0,)), ((), ())),
                                 preferred_element_type=jnp.float32)


def _tc_outer(c0, c1, x, w, b2, t_sp):
    blk = 5120
    cspec = pl.BlockSpec((1, blk), lambda i, t_ref: (0, i))
    grid_spec = pltpu.PrefetchScalarGridSpec(
        num_scalar_prefetch=1,
        grid=(NPAD // blk,),
        in_specs=[
            cspec,
            cspec,
            pl.BlockSpec((8, DF), lambda i, t_ref: (t_ref[0] // 8, 0)),
            pl.BlockSpec((DF, DF), lambda i, t_ref: (0, 0)),
            pl.BlockSpec((1, DF), lambda i, t_ref: (0, 0)),
        ],
        out_specs=pl.BlockSpec((blk, DF), lambda i, t_ref: (i, 0)),
    )
    return pl.pallas_call(
        _tc_body,
        grid_spec=grid_spec,
        out_shape=jax.ShapeDtypeStruct((N, DF), jnp.float32),
    )(t_sp, c0, c1, x, w, b2)


def kernel(x, edge_index, batch_index, W, b):
    c0, c1, t_vec = _sc_counts(edge_index)
    return _tc_outer(c0.reshape(1, NPAD), c1.reshape(1, NPAD), x, W,
                     b.reshape(1, DF), t_vec[:1])
